# speedup vs baseline: 1.4048x; 1.4048x over previous
import functools

import jax
import jax.numpy as jnp
from jax import lax
from jax.experimental import pallas as pl
from jax.experimental.pallas import tpu as pltpu
from jax.experimental.pallas import tpu_sc as plsc

_NC = 2
_NS = 16
_NW = _NC * _NS

_B, _T, _D = 4, 8192, 1024
_K = _T // 2
_ROWS = _B * _K
_RPW = _ROWS // _NW
_CH = 32
_NCHUNK = _RPW // _CH


def _gather_body(x_hbm, idx_hbm, out_hbm, idx_v, buf0, buf1, sem0, sem1):
    wid = lax.axis_index("s") * _NC + lax.axis_index("c")
    base = wid * _RPW
    pltpu.sync_copy(idx_hbm.at[wid], idx_v)

    bufs = (buf0, buf1)
    sems = (sem0, sem1)
    d0 = pltpu.async_copy(x_hbm.at[idx_v.at[0]], bufs[0], sems[0])
    descs = [d0, None]
    for c in range(_NCHUNK):
        descs[c % 2].wait()
        if c + 1 < _NCHUNK:
            descs[(c + 1) % 2] = pltpu.async_copy(
                x_hbm.at[idx_v.at[c + 1]], bufs[(c + 1) % 2], sems[(c + 1) % 2]
            )
        pltpu.sync_copy(bufs[c % 2], out_hbm.at[pl.ds(base + c * _CH, _CH)])


def _sc_gather(x_flat, idx_chunked):
    mesh = plsc.VectorSubcoreMesh(
        core_axis_name="c", subcore_axis_name="s", num_cores=_NC, num_subcores=_NS
    )
    return pl.kernel(
        _gather_body,
        out_type=jax.ShapeDtypeStruct((_ROWS, _D), jnp.float32),
        mesh=mesh,
        scratch_types=[
            pltpu.VMEM((_NCHUNK, _CH), jnp.int32),
            pltpu.VMEM((_CH, _D), jnp.float32),
            pltpu.VMEM((_CH, _D), jnp.float32),
            pltpu.SemaphoreType.DMA,
            pltpu.SemaphoreType.DMA,
        ],
    )(x_flat, idx_chunked)


def kernel(x, rand_scores):
    B, T, D = x.shape
    num_keep = _K
    _, token_indices_keep = jax.lax.top_k(rand_scores, num_keep)
    gidx = token_indices_keep + (jnp.arange(B, dtype=jnp.int32) * T)[:, None]
    gidx = gidx.reshape(_NW, _NCHUNK, _CH)
    out = _sc_gather(x.reshape(B * T, D), gidx)
    return (out.reshape(B, num_keep, D), token_indices_keep)
